# baseline (speedup 1.0000x reference)
import functools

import jax
import jax.numpy as jnp
from jax import lax
from jax.experimental import pallas as pl
from jax.experimental.pallas import tpu as pltpu

_E = 8
_K = 2
_FT = 512


def _ffn_body(xe_ref, w1a_ref, w1b_ref, w2_ref, w_ref, out_ref, acc_ref):
    f = pl.program_id(1)
    xe = xe_ref[...]
    w1a = w1a_ref[0].astype(jnp.bfloat16)
    w1b = w1b_ref[0].astype(jnp.bfloat16)
    dn = (((1,), (1,)), ((), ()))
    h1 = lax.dot_general(xe, w1a, dn, preferred_element_type=jnp.float32)
    h2 = lax.dot_general(xe, w1b, dn, preferred_element_type=jnp.float32)
    act = h1 * (h2 * jax.nn.sigmoid(h2))
    act = act * w_ref[0, 0][:, None]
    contrib = lax.dot_general(act.astype(jnp.bfloat16),
                              w2_ref[0].astype(jnp.bfloat16), dn,
                              preferred_element_type=jnp.float32)

    @pl.when(f == 0)
    def _init():
        acc_ref[...] = contrib

    @pl.when(f > 0)
    def _acc():
        acc_ref[...] += contrib

    @pl.when(f == pl.num_programs(1) - 1)
    def _emit():
        out_ref[0] = acc_ref[...].astype(jnp.bfloat16)


def _expert_ffn(xe, w_sel, W1, W2):
    E, C, D = xe.shape
    d_ff = W2.shape[2]
    nf = d_ff // _FT
    xe2 = xe.reshape(E * C, D)
    grid = (E, nf)
    return pl.pallas_call(
        _ffn_body,
        grid=grid,
        in_specs=[
            pl.BlockSpec((C, D), lambda e, f: (e, 0)),
            pl.BlockSpec((1, _FT, D), lambda e, f: (e, f, 0)),
            pl.BlockSpec((1, _FT, D), lambda e, f: (e, nf + f, 0)),
            pl.BlockSpec((1, D, _FT), lambda e, f: (e, 0, f)),
            pl.BlockSpec((1, 1, C), lambda e, f: (e, 0, 0)),
        ],
        out_specs=pl.BlockSpec((1, C, D), lambda e, f: (e, 0, 0)),
        out_shape=jax.ShapeDtypeStruct((E, C, D), jnp.bfloat16),
        scratch_shapes=[pltpu.VMEM((C, D), jnp.float32)],
        compiler_params=pltpu.CompilerParams(
            vmem_limit_bytes=100 * 1024 * 1024),
    )(xe2, W1, W1, W2, w_sel.reshape(E, 1, C))


@functools.partial(jax.jit, static_argnames=())
def kernel(x, Wg, W1, W2):
    b, s, d = x.shape
    T = b * s
    C = T // _E
    flat_x = x.reshape(T, d)

    gate_logits = flat_x @ Wg.T
    gate_scores = jax.nn.softmax(gate_logits, axis=-1)
    topk_vals, topk_idx = lax.top_k(gate_scores, _K)
    phi_mean = gate_scores.mean(axis=0)
    aux_loss = _E * jnp.sum(phi_mean * phi_mean)

    sel = (topk_idx[:, :, None] == jnp.arange(_E)[None, None, :]).any(axis=1)
    score = jnp.where(sel.T, gate_scores.T, -jnp.inf)
    sel_w, sel_pos = lax.top_k(score, C)
    w_sel = jnp.where(jnp.isfinite(sel_w), sel_w, 0.0)

    flat_bf = flat_x.astype(jnp.bfloat16)
    xe = flat_bf[sel_pos.reshape(-1)].reshape(_E, C, d)
    out = _expert_ffn(xe, w_sel, W1, W2)

    rows1 = (jnp.arange(_E * C, dtype=jnp.int32) + 1).reshape(_E, C)
    rowmap = jnp.zeros((_E, T), jnp.int32).at[
        jnp.arange(_E, dtype=jnp.int32)[:, None], sel_pos].set(
        rows1, unique_indices=True, indices_are_sorted=False)
    rk = jnp.take_along_axis(rowmap.T, topk_idx, axis=1)
    m = (rk > 0).astype(jnp.float32)
    g = out.reshape(_E * C, d)[jnp.maximum(rk - 1, 0).reshape(-1)]
    g = g.reshape(T, _K, d).astype(jnp.float32)
    combined = jnp.einsum("tk,tkd->td", m, g)
    return combined.reshape(b, s, d), aux_loss


# speedup vs baseline: 1.6983x; 1.6983x over previous
import functools

import jax
import jax.numpy as jnp
from jax import lax
from jax.experimental import pallas as pl
from jax.experimental.pallas import tpu as pltpu

_E = 8
_K = 2
_FT = 1024


def _ffn_body(xe_ref, w1a_ref, w1b_ref, w2_ref, w_ref, out_ref):
    f = pl.program_id(1)
    xe = xe_ref[...]
    w1a = w1a_ref[0].astype(jnp.bfloat16)
    w1b = w1b_ref[0].astype(jnp.bfloat16)
    dn = (((1,), (1,)), ((), ()))
    h1 = lax.dot_general(xe, w1a, dn, preferred_element_type=jnp.float32)
    h2 = lax.dot_general(xe, w1b, dn, preferred_element_type=jnp.float32)
    act = h1 * (h2 * jax.nn.sigmoid(h2))
    act = act * w_ref[0, 0][:, None]
    contrib = lax.dot_general(act.astype(jnp.bfloat16),
                              w2_ref[0].astype(jnp.bfloat16), dn,
                              preferred_element_type=jnp.float32)

    @pl.when(f == 0)
    def _init():
        out_ref[0] = contrib

    @pl.when(f > 0)
    def _acc():
        out_ref[0] += contrib


def _expert_ffn(xe, w_sel, W1, W2):
    E, C, D = xe.shape
    d_ff = W2.shape[2]
    nf = d_ff // _FT
    xe2 = xe.reshape(E * C, D)
    grid = (E, nf)
    return pl.pallas_call(
        _ffn_body,
        grid=grid,
        in_specs=[
            pl.BlockSpec((C, D), lambda e, f: (e, 0)),
            pl.BlockSpec((1, _FT, D), lambda e, f: (e, f, 0)),
            pl.BlockSpec((1, _FT, D), lambda e, f: (e, nf + f, 0)),
            pl.BlockSpec((1, D, _FT), lambda e, f: (e, 0, f)),
            pl.BlockSpec((1, 1, C), lambda e, f: (e, 0, 0)),
        ],
        out_specs=pl.BlockSpec((1, C, D), lambda e, f: (e, 0, 0)),
        out_shape=jax.ShapeDtypeStruct((E, C, D), jnp.float32),
        compiler_params=pltpu.CompilerParams(
            vmem_limit_bytes=100 * 1024 * 1024),
    )(xe2, W1, W1, W2, w_sel.reshape(E, 1, C))


@functools.partial(jax.jit, static_argnames=())
def kernel(x, Wg, W1, W2):
    b, s, d = x.shape
    T = b * s
    C = T // _E
    flat_x = x.reshape(T, d)

    gate_logits = flat_x @ Wg.T
    gate_scores = jax.nn.softmax(gate_logits, axis=-1)
    topk_vals, topk_idx = lax.top_k(gate_scores, _K)
    phi_mean = gate_scores.mean(axis=0)
    aux_loss = _E * jnp.sum(phi_mean * phi_mean)

    sel = (topk_idx[:, :, None] == jnp.arange(_E)[None, None, :]).any(axis=1)
    score = jnp.where(sel.T, gate_scores.T, -jnp.inf)
    sel_w, sel_pos = lax.top_k(score, C)
    w_sel = jnp.where(jnp.isfinite(sel_w), sel_w, 0.0)

    flat_bf = flat_x.astype(jnp.bfloat16)
    xe = flat_bf[sel_pos.reshape(-1)].reshape(_E, C, d)
    out = _expert_ffn(xe, w_sel, W1, W2)
    combined = jnp.zeros((T, d), jnp.float32).at[sel_pos.reshape(-1)].add(
        out.reshape(_E * C, d))
    return combined.reshape(b, s, d), aux_loss


# speedup vs baseline: 1.7531x; 1.0323x over previous
import functools

import jax
import jax.numpy as jnp
from jax import lax
from jax.experimental import pallas as pl
from jax.experimental.pallas import tpu as pltpu

_E = 8
_K = 2
_FT = 2048


def _ffn_body(xe_ref, w1a_ref, w1b_ref, w2_ref, w_ref, out_ref):
    f = pl.program_id(1)
    xe = xe_ref[...]
    w1a = w1a_ref[0].astype(jnp.bfloat16)
    w1b = w1b_ref[0].astype(jnp.bfloat16)
    dn = (((1,), (1,)), ((), ()))
    h1 = lax.dot_general(xe, w1a, dn, preferred_element_type=jnp.float32)
    h2 = lax.dot_general(xe, w1b, dn, preferred_element_type=jnp.float32)
    act = h1 * (h2 * jax.nn.sigmoid(h2))
    act = act * w_ref[0, 0][:, None]
    contrib = lax.dot_general(act.astype(jnp.bfloat16),
                              w2_ref[0].astype(jnp.bfloat16), dn,
                              preferred_element_type=jnp.float32)

    @pl.when(f == 0)
    def _init():
        out_ref[0] = contrib

    @pl.when(f > 0)
    def _acc():
        out_ref[0] += contrib


def _expert_ffn(xe, w_sel, W1, W2):
    E, C, D = xe.shape
    d_ff = W2.shape[2]
    nf = d_ff // _FT
    xe2 = xe.reshape(E * C, D)
    grid = (E, nf)
    return pl.pallas_call(
        _ffn_body,
        grid=grid,
        in_specs=[
            pl.BlockSpec((C, D), lambda e, f: (e, 0)),
            pl.BlockSpec((1, _FT, D), lambda e, f: (e, f, 0)),
            pl.BlockSpec((1, _FT, D), lambda e, f: (e, nf + f, 0)),
            pl.BlockSpec((1, D, _FT), lambda e, f: (e, 0, f)),
            pl.BlockSpec((1, 1, C), lambda e, f: (e, 0, 0)),
        ],
        out_specs=pl.BlockSpec((1, C, D), lambda e, f: (e, 0, 0)),
        out_shape=jax.ShapeDtypeStruct((E, C, D), jnp.float32),
        compiler_params=pltpu.CompilerParams(
            vmem_limit_bytes=100 * 1024 * 1024),
    )(xe2, W1, W1, W2, w_sel.reshape(E, 1, C))


@functools.partial(jax.jit, static_argnames=())
def kernel(x, Wg, W1, W2):
    b, s, d = x.shape
    T = b * s
    C = T // _E
    flat_x = x.reshape(T, d)

    gate_logits = flat_x @ Wg.T
    gate_scores = jax.nn.softmax(gate_logits, axis=-1)
    topk_vals, topk_idx = lax.top_k(gate_scores, _K)
    phi_mean = gate_scores.mean(axis=0)
    aux_loss = _E * jnp.sum(phi_mean * phi_mean)

    sel = (topk_idx[:, :, None] == jnp.arange(_E)[None, None, :]).any(axis=1)
    score = jnp.where(sel.T, gate_scores.T, -jnp.inf)
    sel_w, sel_pos = lax.top_k(score, C)
    w_sel = jnp.where(jnp.isfinite(sel_w), sel_w, 0.0)

    flat_bf = flat_x.astype(jnp.bfloat16)
    xe = flat_bf[sel_pos.reshape(-1)].reshape(_E, C, d)
    out = _expert_ffn(xe, w_sel, W1, W2)
    combined = jnp.zeros((T, d), jnp.float32).at[sel_pos.reshape(-1)].add(
        out.reshape(_E * C, d))
    return combined.reshape(b, s, d), aux_loss
